# baseline (speedup 1.0000x reference)
import jax
import jax.numpy as jnp
from jax import lax
from jax.experimental import pallas as pl
from jax.experimental.pallas import tpu as pltpu
from jax.experimental.pallas import tpu_sc as plsc

_K = 64
_BN = 4096
_NLANE = 16
_NCORE = 2
_CHUNK = 4
_UNROLL = 8


def _enc_body(x_ref, w_ref, b_ref, out_ref):
    xs = x_ref[:, 0, :] + x_ref[:, 1, :]
    out_ref[...] = (
        jnp.dot(xs, w_ref[...], preferred_element_type=jnp.float32) + b_ref[...]
    )


def _make_sc_kernel(B, S, T, L, D):
    DM = T * L * D
    SV = S // _NLANE
    DV = D // _NLANE
    NCHUNK = _K // _CHUNK
    KV = _K // _NLANE

    mesh = plsc.VectorSubcoreMesh(core_axis_name="c", subcore_axis_name="s")

    def body(bits_hbm, pre128_hbm, x_hbm, wdec_hbm, bdec_hbm,
             zbits_hbm, xhat_hbm, loss_hbm,
             bits_v, cand_v, hist_v, sel_v, sel8_v, selbit_v, rowid_v, vrows_v,
             selval_v, gbuf_v, acc_v, xrow_v, bdec_v, loss_v,
             sem0, sem1, semg):
        cid = lax.axis_index("c")
        sid = lax.axis_index("s")
        b = sid * _NCORE + cid

        iota16 = lax.broadcasted_iota(jnp.int32, (_NLANE,), 0)
        ones16f = jnp.ones((_NLANE,), jnp.float32)
        zeros16i = jnp.zeros((_NLANE,), jnp.int32)
        zeros16f = jnp.zeros((_NLANE,), jnp.float32)

        d_pre = pltpu.async_copy(
            bits_hbm.at[pl.ds(pl.multiple_of(b * S, 8), S)], bits_v, sem0)
        d_x = pltpu.async_copy(
            x_hbm.at[pl.ds(pl.multiple_of(b * DM, 8), DM)], xrow_v, sem1)
        d_bd = pltpu.async_copy(bdec_hbm, bdec_v, semg)

        def zero_hist(i, _):
            for u in range(_UNROLL):
                hist_v[pl.ds((i * _UNROLL + u) * _NLANE, _NLANE)] = zeros16f
            return 0

        def byte_of(bits, shift):
            key = bits ^ (lax.shift_right_arithmetic(bits, 31)
                          & jnp.int32(0x7FFFFFFF))
            byte = lax.shift_right_logical(key, shift) & jnp.int32(0xFF)
            if shift == 24:
                byte = byte ^ jnp.int32(0x80)
            return byte

        def find_bin(need):
            need_f = need.astype(jnp.float32)

            def gscan(i, st):
                cum, gstar, gabove = st
                g = 15 - i
                acc = zeros16f
                for u in range(16):
                    acc = acc + hist_v[pl.ds(g * 256 + u * _NLANE, _NLANE)]
                cnt = jnp.sum(acc)
                hit = jnp.logical_and(gstar < 0, cum + cnt >= need_f)
                return (cum + cnt,
                        jnp.where(hit, g, gstar),
                        jnp.where(hit, cum, gabove))
            _, gstar, gabove = lax.fori_loop(
                0, 16, gscan,
                (jnp.float32(0), jnp.int32(-1), jnp.float32(0)))

            def bscan(i, st):
                cum, bstar, above = st
                j = gstar * 16 + (15 - i)
                cnt = jnp.sum(hist_v[pl.ds(j * _NLANE, _NLANE)])
                hit = jnp.logical_and(bstar < 0, cum + cnt >= need_f)
                return (cum + cnt,
                        jnp.where(hit, j, bstar),
                        jnp.where(hit, cum, above))
            _, bstar, above = lax.fori_loop(
                0, 16, bscan, (gabove, jnp.int32(-1), jnp.float32(0)))
            return bstar, above.astype(jnp.int32)

        lax.fori_loop(0, 256 // _UNROLL, zero_hist, 0)
        d_pre.wait()

        def hist0(i, _):
            for u in range(_UNROLL):
                bits = bits_v[pl.ds((i * _UNROLL + u) * _NLANE, _NLANE)]
                byte = byte_of(bits, 24)
                plsc.addupdate_scatter(hist_v, [byte * _NLANE + iota16], ones16f)
            return 0
        lax.fori_loop(0, SV // _UNROLL, hist0, 0)

        need = jnp.int32(_K)
        bstar, above = find_bin(need)

        def collect0(i, st):
            ncand, selcnt = st
            for u in range(_UNROLL):
                bits = bits_v[pl.ds((i * _UNROLL + u) * _NLANE, _NLANE)]
                byte = byte_of(bits, 24)
                idx = (i * _UNROLL + u) * _NLANE + iota16
                m_gt = byte > bstar
                plsc.store_compressed(sel_v.at[pl.ds(selcnt, _NLANE)], idx,
                                      mask=m_gt)
                selcnt = selcnt + plsc.all_reduce_population_count(m_gt)[0]
                m_eq = byte == bstar
                plsc.store_compressed(cand_v.at[pl.ds(ncand, _NLANE)], idx,
                                      mask=m_eq)
                ncand = ncand + plsc.all_reduce_population_count(m_eq)[0]
            return (ncand, selcnt)
        n, selcnt = lax.fori_loop(0, SV // _UNROLL, collect0,
                                  (jnp.int32(0), jnp.int32(0)))
        need = need - above

        for shift in (16, 8, 0):
            lax.fori_loop(0, 256 // _UNROLL, zero_hist, 0)
            nv = (n + _UNROLL * _NLANE - 1) // (_UNROLL * _NLANE)

            def histl(i, _, shift=shift, n=n):
                for u in range(_UNROLL):
                    base = (i * _UNROLL + u) * _NLANE
                    idx = cand_v[pl.ds(base, _NLANE)]
                    lanemask = (base + iota16) < n
                    bits = plsc.load_gather(bits_v, [idx], mask=lanemask)
                    byte = byte_of(bits, shift)
                    plsc.addupdate_scatter(hist_v, [byte * _NLANE + iota16],
                                           ones16f, mask=lanemask)
                return 0
            lax.fori_loop(0, nv, histl, 0)

            bstar, above = find_bin(need)

            def collectl(i, st, shift=shift, n=n, bstar=bstar):
                ncand, selcnt = st
                for u in range(_UNROLL):
                    base = (i * _UNROLL + u) * _NLANE
                    idx = cand_v[pl.ds(base, _NLANE)]
                    lanemask = (base + iota16) < n
                    bits = plsc.load_gather(bits_v, [idx], mask=lanemask)
                    byte = byte_of(bits, shift)
                    m_gt = jnp.logical_and(byte > bstar, lanemask)
                    plsc.store_compressed(sel_v.at[pl.ds(selcnt, _NLANE)], idx,
                                          mask=m_gt)
                    selcnt = selcnt + plsc.all_reduce_population_count(m_gt)[0]
                    m_eq = jnp.logical_and(byte == bstar, lanemask)
                    plsc.store_compressed(cand_v.at[pl.ds(ncand, _NLANE)], idx,
                                          mask=m_eq)
                    ncand = ncand + plsc.all_reduce_population_count(m_eq)[0]
                return (ncand, selcnt)
            n, selcnt = lax.fori_loop(0, nv, collectl,
                                      (jnp.int32(0), selcnt))
            need = need - above

        for i in range(KV):
            idx = cand_v[pl.ds(i * _NLANE, _NLANE)]
            pos = i * _NLANE + iota16
            m = pos < need
            plsc.store_compressed(sel_v.at[pl.ds(selcnt, _NLANE)], idx, mask=m)
            selcnt = selcnt + plsc.all_reduce_population_count(m)[0]

        for i in range(KV):
            idx = sel_v[pl.ds(i * _NLANE, _NLANE)]
            bits = plsc.load_gather(bits_v, [idx])
            selbit_v[pl.ds(i * _NLANE, _NLANE)] = jnp.maximum(bits, 0)
            g = b * S + idx
            rowid_v[pl.ds(i * _NLANE, _NLANE)] = lax.shift_right_logical(g, 7)
            pos = i * _NLANE + iota16
            dst = lax.shift_right_logical(pos, 2) * 8 + (pos & jnp.int32(3))
            plsc.store_scatter(sel8_v, [dst], idx)

        d_vals = pltpu.async_copy(pre128_hbm.at[rowid_v], vrows_v, semg)

        def zero_row(i, _):
            for u in range(_UNROLL):
                bits_v[pl.ds((i * _UNROLL + u) * _NLANE, _NLANE)] = zeros16i
            return 0
        lax.fori_loop(0, SV // _UNROLL, zero_row, 0)

        for i in range(KV):
            idx = sel_v[pl.ds(i * _NLANE, _NLANE)]
            zb = selbit_v[pl.ds(i * _NLANE, _NLANE)]
            plsc.store_scatter(bits_v, [idx], zb)
        pltpu.sync_copy(bits_v, zbits_hbm.at[pl.ds(pl.multiple_of(b * S, 8), S)])

        d_bd.wait()
        d_vals.wait()

        for i in range(KV):
            idx = sel_v[pl.ds(i * _NLANE, _NLANE)]
            col = (b * S + idx) & jnp.int32(127)
            v = plsc.load_gather(vrows_v, [i * _NLANE + iota16, col])
            selval_v[pl.ds(i * _NLANE, _NLANE)] = jnp.maximum(v, 0.0)

        def init_acc(j, _):
            for u in range(_UNROLL):
                o = (j * _UNROLL + u) * _NLANE
                acc_v[pl.ds(o, _NLANE)] = bdec_v[pl.ds(o, _NLANE)]
            return 0
        lax.fori_loop(0, DM // _NLANE // _UNROLL, init_acc, 0)

        sems = (sem0, sem1)
        descs = [None] * NCHUNK
        descs[0] = pltpu.async_copy(
            wdec_hbm.at[sel8_v.at[pl.ds(0, _CHUNK)]], gbuf_v.at[0], sems[0])
        for c in range(NCHUNK):
            if c + 1 < NCHUNK:
                descs[c + 1] = pltpu.async_copy(
                    wdec_hbm.at[sel8_v.at[pl.ds((c + 1) * 8, _CHUNK)]],
                    gbuf_v.at[(c + 1) % 2], sems[(c + 1) % 2])
            descs[c].wait()
            svals = []
            for r in range(_CHUNK):
                kg = c * _CHUNK + r
                vv = selval_v[pl.ds((kg // _NLANE) * _NLANE, _NLANE)]
                svals.append(vv[kg % _NLANE])

            def acc_fn(j, _, c=c, svals=svals):
                for t in range(T):
                    for l in range(L):
                        off = (t * L + l) * D
                        a = acc_v[pl.ds(off + j * _NLANE, _NLANE)]
                        for r in range(_CHUNK):
                            a = a + (gbuf_v[c % 2, r, t, l, pl.ds(j * _NLANE, _NLANE)]
                                     * svals[r])
                        acc_v[pl.ds(off + j * _NLANE, _NLANE)] = a
                return 0
            lax.fori_loop(0, DV, acc_fn, 0)

        pltpu.sync_copy(acc_v, xhat_hbm.at[pl.ds(pl.multiple_of(b * DM, 8), DM)])

        d_x.wait()

        def loss_fn(j, acc):
            for u in range(_UNROLL):
                o = (j * _UNROLL + u) * _NLANE
                d = acc_v[pl.ds(o, _NLANE)] - xrow_v[pl.ds(o, _NLANE)]
                acc = acc + d * d
            return acc
        lvec = lax.fori_loop(0, DM // _NLANE // _UNROLL, loss_fn, zeros16f)
        loss_v[...] = lvec
        pltpu.sync_copy(
            loss_v, loss_hbm.at[pl.ds(pl.multiple_of(b * _NLANE, 8), _NLANE)])

    sck = pl.kernel(
        body,
        out_type=(
            jax.ShapeDtypeStruct((B * S,), jnp.int32),
            jax.ShapeDtypeStruct((B * DM,), jnp.float32),
            jax.ShapeDtypeStruct((B * _NLANE,), jnp.float32),
        ),
        mesh=mesh,
        compiler_params=pltpu.CompilerParams(needs_layout_passes=False),
        scratch_types=[
            pltpu.VMEM((S,), jnp.int32),
            pltpu.VMEM((S + _NLANE,), jnp.int32),
            pltpu.VMEM((256 * _NLANE,), jnp.float32),
            pltpu.VMEM((_K + _NLANE,), jnp.int32),
            pltpu.VMEM((2 * _K,), jnp.int32),
            pltpu.VMEM((_K + _NLANE,), jnp.int32),
            pltpu.VMEM((_K,), jnp.int32),
            pltpu.VMEM((_K, 128), jnp.float32),
            pltpu.VMEM((_K + _NLANE,), jnp.float32),
            pltpu.VMEM((2, _CHUNK, T, L, D), jnp.float32),
            pltpu.VMEM((DM,), jnp.float32),
            pltpu.VMEM((DM,), jnp.float32),
            pltpu.VMEM((DM,), jnp.float32),
            pltpu.VMEM((_NLANE,), jnp.float32),
            pltpu.SemaphoreType.DMA,
            pltpu.SemaphoreType.DMA,
            pltpu.SemaphoreType.DMA,
        ],
    )
    return sck


def kernel(x, W_enc, b_enc, W_dec, b_dec):
    B, T, L, D = x.shape
    S = W_enc.shape[-1]
    KD = L * D
    DM = T * L * D
    x2 = x.reshape(B, T, KD)
    w2 = W_enc.reshape(KD, S)
    pre = pl.pallas_call(
        _enc_body,
        grid=(S // _BN,),
        in_specs=[
            pl.BlockSpec((B, T, KD), lambda j: (0, 0, 0)),
            pl.BlockSpec((KD, _BN), lambda j: (0, j)),
            pl.BlockSpec((1, _BN), lambda j: (0, j)),
        ],
        out_specs=pl.BlockSpec((B, _BN), lambda j: (0, j)),
        out_shape=jax.ShapeDtypeStruct((B, S), jnp.float32),
    )(x2, w2, b_enc.reshape(1, S))

    pre_bits = lax.bitcast_convert_type(pre, jnp.int32).reshape(B * S)
    pre128 = pre.reshape(B * S // 128, 128)

    sck = _make_sc_kernel(B, S, T, L, D)
    zbits_flat, xhat_flat, loss_part = sck(
        pre_bits, pre128, x.reshape(B * DM), W_dec, b_dec.reshape(DM))

    z = lax.bitcast_convert_type(zbits_flat, jnp.float32).reshape(B, S)
    x_hat = xhat_flat.reshape(B, T, L, D)
    recon = jnp.sum(loss_part) / jnp.float32(B * T * L)
    return (recon, x_hat, z)


# speedup vs baseline: 1.0112x; 1.0112x over previous
import jax
import jax.numpy as jnp
from jax import lax
from jax.experimental import pallas as pl
from jax.experimental.pallas import tpu as pltpu
from jax.experimental.pallas import tpu_sc as plsc

_K = 64
_BN = 2048
_NLANE = 16
_NCORE = 2
_CHUNK = 4
_UNROLL = 4


def _enc_body(x_ref, w_ref, b_ref, out_ref):
    xs = x_ref[:, 0, :] + x_ref[:, 1, :]
    out_ref[...] = (
        jnp.dot(xs, w_ref[...], preferred_element_type=jnp.float32) + b_ref[...]
    )


def _make_sc_kernel(B, S, T, L, D):
    DM = T * L * D
    SV = S // _NLANE
    DV = D // _NLANE
    NCHUNK = _K // _CHUNK
    KV = _K // _NLANE

    mesh = plsc.VectorSubcoreMesh(core_axis_name="c", subcore_axis_name="s")

    def body(bits_hbm, pre128_hbm, x_hbm, wdec_hbm, bdec_hbm,
             zbits_hbm, xhat_hbm, loss_hbm,
             bits_v, cand_v, hist_v, sel_v, sel8_v, selbit_v, rowid_v, vrows_v,
             selval_v, gbuf_v, acc_v, xrow_v, bdec_v, loss_v,
             sem0, sem1, semg):
        cid = lax.axis_index("c")
        sid = lax.axis_index("s")
        b = sid * _NCORE + cid

        iota16 = lax.broadcasted_iota(jnp.int32, (_NLANE,), 0)
        ones16f = jnp.ones((_NLANE,), jnp.float32)
        zeros16i = jnp.zeros((_NLANE,), jnp.int32)
        zeros16f = jnp.zeros((_NLANE,), jnp.float32)

        d_pre = pltpu.async_copy(
            bits_hbm.at[pl.ds(pl.multiple_of(b * S, 8), S)], bits_v, sem0)
        d_x = pltpu.async_copy(
            x_hbm.at[pl.ds(pl.multiple_of(b * DM, 8), DM)], xrow_v, sem1)
        d_bd = pltpu.async_copy(bdec_hbm, bdec_v, semg)

        def zero_hist(i, _):
            for u in range(_UNROLL):
                hist_v[pl.ds((i * _UNROLL + u) * _NLANE, _NLANE)] = zeros16f
            return 0

        def byte_of(bits, shift):
            key = bits ^ (lax.shift_right_arithmetic(bits, 31)
                          & jnp.int32(0x7FFFFFFF))
            byte = lax.shift_right_logical(key, shift) & jnp.int32(0xFF)
            if shift == 24:
                byte = byte ^ jnp.int32(0x80)
            return byte

        def find_bin(need):
            need_f = need.astype(jnp.float32)

            def gscan(i, st):
                cum, gstar, gabove = st
                g = 15 - i
                acc = zeros16f
                for u in range(16):
                    acc = acc + hist_v[pl.ds(g * 256 + u * _NLANE, _NLANE)]
                cnt = jnp.sum(acc)
                hit = jnp.logical_and(gstar < 0, cum + cnt >= need_f)
                return (cum + cnt,
                        jnp.where(hit, g, gstar),
                        jnp.where(hit, cum, gabove))
            _, gstar, gabove = lax.fori_loop(
                0, 16, gscan,
                (jnp.float32(0), jnp.int32(-1), jnp.float32(0)))

            def bscan(i, st):
                cum, bstar, above = st
                j = gstar * 16 + (15 - i)
                cnt = jnp.sum(hist_v[pl.ds(j * _NLANE, _NLANE)])
                hit = jnp.logical_and(bstar < 0, cum + cnt >= need_f)
                return (cum + cnt,
                        jnp.where(hit, j, bstar),
                        jnp.where(hit, cum, above))
            _, bstar, above = lax.fori_loop(
                0, 16, bscan, (gabove, jnp.int32(-1), jnp.float32(0)))
            return bstar, above.astype(jnp.int32)

        lax.fori_loop(0, 256 // _UNROLL, zero_hist, 0)
        d_pre.wait()

        def hist0(i, _):
            for u in range(_UNROLL):
                bits = bits_v[pl.ds((i * _UNROLL + u) * _NLANE, _NLANE)]
                byte = byte_of(bits, 24)
                plsc.addupdate_scatter(hist_v, [byte * _NLANE + iota16], ones16f)
            return 0
        lax.fori_loop(0, SV // _UNROLL, hist0, 0)

        need = jnp.int32(_K)
        bstar, above = find_bin(need)

        def collect0(i, st):
            ncand, selcnt = st
            for u in range(_UNROLL):
                bits = bits_v[pl.ds((i * _UNROLL + u) * _NLANE, _NLANE)]
                byte = byte_of(bits, 24)
                idx = (i * _UNROLL + u) * _NLANE + iota16
                m_gt = byte > bstar
                plsc.store_compressed(sel_v.at[pl.ds(selcnt, _NLANE)], idx,
                                      mask=m_gt)
                selcnt = selcnt + plsc.all_reduce_population_count(m_gt)[0]
                m_eq = byte == bstar
                plsc.store_compressed(cand_v.at[pl.ds(ncand, _NLANE)], idx,
                                      mask=m_eq)
                ncand = ncand + plsc.all_reduce_population_count(m_eq)[0]
            return (ncand, selcnt)
        n, selcnt = lax.fori_loop(0, SV // _UNROLL, collect0,
                                  (jnp.int32(0), jnp.int32(0)))
        need = need - above

        for shift in (16, 8, 0):
            lax.fori_loop(0, 256 // _UNROLL, zero_hist, 0)
            nv = (n + _UNROLL * _NLANE - 1) // (_UNROLL * _NLANE)

            def histl(i, _, shift=shift, n=n):
                for u in range(_UNROLL):
                    base = (i * _UNROLL + u) * _NLANE
                    idx = cand_v[pl.ds(base, _NLANE)]
                    lanemask = (base + iota16) < n
                    bits = plsc.load_gather(bits_v, [idx], mask=lanemask)
                    byte = byte_of(bits, shift)
                    plsc.addupdate_scatter(hist_v, [byte * _NLANE + iota16],
                                           ones16f, mask=lanemask)
                return 0
            lax.fori_loop(0, nv, histl, 0)

            bstar, above = find_bin(need)

            def collectl(i, st, shift=shift, n=n, bstar=bstar):
                ncand, selcnt = st
                for u in range(_UNROLL):
                    base = (i * _UNROLL + u) * _NLANE
                    idx = cand_v[pl.ds(base, _NLANE)]
                    lanemask = (base + iota16) < n
                    bits = plsc.load_gather(bits_v, [idx], mask=lanemask)
                    byte = byte_of(bits, shift)
                    m_gt = jnp.logical_and(byte > bstar, lanemask)
                    plsc.store_compressed(sel_v.at[pl.ds(selcnt, _NLANE)], idx,
                                          mask=m_gt)
                    selcnt = selcnt + plsc.all_reduce_population_count(m_gt)[0]
                    m_eq = jnp.logical_and(byte == bstar, lanemask)
                    plsc.store_compressed(cand_v.at[pl.ds(ncand, _NLANE)], idx,
                                          mask=m_eq)
                    ncand = ncand + plsc.all_reduce_population_count(m_eq)[0]
                return (ncand, selcnt)
            n, selcnt = lax.fori_loop(0, nv, collectl,
                                      (jnp.int32(0), selcnt))
            need = need - above

        for i in range(KV):
            idx = cand_v[pl.ds(i * _NLANE, _NLANE)]
            pos = i * _NLANE + iota16
            m = pos < need
            plsc.store_compressed(sel_v.at[pl.ds(selcnt, _NLANE)], idx, mask=m)
            selcnt = selcnt + plsc.all_reduce_population_count(m)[0]

        for i in range(KV):
            idx = sel_v[pl.ds(i * _NLANE, _NLANE)]
            bits = plsc.load_gather(bits_v, [idx])
            selbit_v[pl.ds(i * _NLANE, _NLANE)] = jnp.maximum(bits, 0)
            g = b * S + idx
            rowid_v[pl.ds(i * _NLANE, _NLANE)] = lax.shift_right_logical(g, 7)
            pos = i * _NLANE + iota16
            dst = lax.shift_right_logical(pos, 2) * 8 + (pos & jnp.int32(3))
            plsc.store_scatter(sel8_v, [dst], idx)

        d_vals = pltpu.async_copy(pre128_hbm.at[rowid_v], vrows_v, semg)

        def zero_row(i, _):
            for u in range(_UNROLL):
                bits_v[pl.ds((i * _UNROLL + u) * _NLANE, _NLANE)] = zeros16i
            return 0
        lax.fori_loop(0, SV // _UNROLL, zero_row, 0)

        for i in range(KV):
            idx = sel_v[pl.ds(i * _NLANE, _NLANE)]
            zb = selbit_v[pl.ds(i * _NLANE, _NLANE)]
            plsc.store_scatter(bits_v, [idx], zb)
        pltpu.sync_copy(bits_v, zbits_hbm.at[pl.ds(pl.multiple_of(b * S, 8), S)])

        d_bd.wait()
        d_vals.wait()

        for i in range(KV):
            idx = sel_v[pl.ds(i * _NLANE, _NLANE)]
            col = (b * S + idx) & jnp.int32(127)
            v = plsc.load_gather(vrows_v, [i * _NLANE + iota16, col])
            selval_v[pl.ds(i * _NLANE, _NLANE)] = jnp.maximum(v, 0.0)

        def init_acc(j, _):
            for u in range(_UNROLL):
                o = (j * _UNROLL + u) * _NLANE
                acc_v[pl.ds(o, _NLANE)] = bdec_v[pl.ds(o, _NLANE)]
            return 0
        lax.fori_loop(0, DM // _NLANE // _UNROLL, init_acc, 0)

        sems = (sem0, sem1)
        descs = [None] * NCHUNK
        descs[0] = pltpu.async_copy(
            wdec_hbm.at[sel8_v.at[pl.ds(0, _CHUNK)]], gbuf_v.at[0], sems[0])
        for c in range(NCHUNK):
            if c + 1 < NCHUNK:
                descs[c + 1] = pltpu.async_copy(
                    wdec_hbm.at[sel8_v.at[pl.ds((c + 1) * 8, _CHUNK)]],
                    gbuf_v.at[(c + 1) % 2], sems[(c + 1) % 2])
            descs[c].wait()
            svals = []
            for r in range(_CHUNK):
                kg = c * _CHUNK + r
                vv = selval_v[pl.ds((kg // _NLANE) * _NLANE, _NLANE)]
                svals.append(vv[kg % _NLANE])

            def acc_fn(j, _, c=c, svals=svals):
                for t in range(T):
                    for l in range(L):
                        off = (t * L + l) * D
                        a = acc_v[pl.ds(off + j * _NLANE, _NLANE)]
                        for r in range(_CHUNK):
                            a = a + (gbuf_v[c % 2, r, t, l, pl.ds(j * _NLANE, _NLANE)]
                                     * svals[r])
                        acc_v[pl.ds(off + j * _NLANE, _NLANE)] = a
                return 0
            lax.fori_loop(0, DV, acc_fn, 0)

        pltpu.sync_copy(acc_v, xhat_hbm.at[pl.ds(pl.multiple_of(b * DM, 8), DM)])

        d_x.wait()

        def loss_fn(j, acc):
            for u in range(_UNROLL):
                o = (j * _UNROLL + u) * _NLANE
                d = acc_v[pl.ds(o, _NLANE)] - xrow_v[pl.ds(o, _NLANE)]
                acc = acc + d * d
            return acc
        lvec = lax.fori_loop(0, DM // _NLANE // _UNROLL, loss_fn, zeros16f)
        loss_v[...] = lvec
        pltpu.sync_copy(
            loss_v, loss_hbm.at[pl.ds(pl.multiple_of(b * _NLANE, 8), _NLANE)])

    sck = pl.kernel(
        body,
        out_type=(
            jax.ShapeDtypeStruct((B * S,), jnp.int32),
            jax.ShapeDtypeStruct((B * DM,), jnp.float32),
            jax.ShapeDtypeStruct((B * _NLANE,), jnp.float32),
        ),
        mesh=mesh,
        compiler_params=pltpu.CompilerParams(needs_layout_passes=False),
        scratch_types=[
            pltpu.VMEM((S,), jnp.int32),
            pltpu.VMEM((S + _NLANE,), jnp.int32),
            pltpu.VMEM((256 * _NLANE,), jnp.float32),
            pltpu.VMEM((_K + _NLANE,), jnp.int32),
            pltpu.VMEM((2 * _K,), jnp.int32),
            pltpu.VMEM((_K + _NLANE,), jnp.int32),
            pltpu.VMEM((_K,), jnp.int32),
            pltpu.VMEM((_K, 128), jnp.float32),
            pltpu.VMEM((_K + _NLANE,), jnp.float32),
            pltpu.VMEM((2, _CHUNK, T, L, D), jnp.float32),
            pltpu.VMEM((DM,), jnp.float32),
            pltpu.VMEM((DM,), jnp.float32),
            pltpu.VMEM((DM,), jnp.float32),
            pltpu.VMEM((_NLANE,), jnp.float32),
            pltpu.SemaphoreType.DMA,
            pltpu.SemaphoreType.DMA,
            pltpu.SemaphoreType.DMA,
        ],
    )
    return sck


def kernel(x, W_enc, b_enc, W_dec, b_dec):
    B, T, L, D = x.shape
    S = W_enc.shape[-1]
    KD = L * D
    DM = T * L * D
    x2 = x.reshape(B, T, KD)
    w2 = W_enc.reshape(KD, S)
    pre = pl.pallas_call(
        _enc_body,
        grid=(S // _BN,),
        in_specs=[
            pl.BlockSpec((B, T, KD), lambda j: (0, 0, 0)),
            pl.BlockSpec((KD, _BN), lambda j: (0, j)),
            pl.BlockSpec((1, _BN), lambda j: (0, j)),
        ],
        out_specs=pl.BlockSpec((B, _BN), lambda j: (0, j)),
        out_shape=jax.ShapeDtypeStruct((B, S), jnp.float32),
    )(x2, w2, b_enc.reshape(1, S))

    pre_bits = lax.bitcast_convert_type(pre, jnp.int32).reshape(B * S)
    pre128 = pre.reshape(B * S // 128, 128)

    sck = _make_sc_kernel(B, S, T, L, D)
    zbits_flat, xhat_flat, loss_part = sck(
        pre_bits, pre128, x.reshape(B * DM), W_dec, b_dec.reshape(DM))

    z = lax.bitcast_convert_type(zbits_flat, jnp.float32).reshape(B, S)
    x_hat = xhat_flat.reshape(B, T, L, D)
    recon = jnp.sum(loss_part) / jnp.float32(B * T * L)
    return (recon, x_hat, z)
